# baseline (speedup 1.0000x reference)
import functools

import jax
import jax.numpy as jnp
from jax import lax
from jax.experimental import pallas as pl
from jax.experimental.pallas import tpu as pltpu
from jax.experimental.pallas import tpu_sc as plsc

N_USERS = 100000
N_ITEMS = 100000
EMBED = 32
HALF = 16

CH = 128
G = 2
NSLOT = 6
AG = 3
AI = 4
ZR = 64
N_EDGE = 1600000
N_CHUNKS = N_EDGE // CH
CHUNKS_PER_TILE = N_CHUNKS // 16
N_MAIN = CHUNKS_PER_TILE - 1
N_GROUPS = N_MAIN // G
ROWS_PER_TILE = 6272
ACC_ROWS = ROWS_PER_TILE * 16
PACK_ROWS = N_USERS // 8


def _sc_segment_sum(edges_il, table):
    mesh = plsc.VectorSubcoreMesh(core_axis_name="c", subcore_axis_name="s")

    @functools.partial(
        pl.kernel,
        out_type=(
            jax.ShapeDtypeStruct((N_USERS, HALF), jnp.float32),
            jax.ShapeDtypeStruct((N_USERS, HALF), jnp.float32),
        ),
        mesh=mesh,
        scratch_types=[
            pltpu.VMEM((NSLOT, G, 2, CH), jnp.int32),
            pltpu.VMEM((NSLOT, G, CH, HALF), jnp.float32),
            pltpu.VMEM((ZR, HALF), jnp.float32),
            pltpu.VMEM_SHARED((ACC_ROWS, HALF), jnp.float32),
            pltpu.SemaphoreType.DMA((NSLOT,)),
            pltpu.SemaphoreType.DMA((NSLOT,)),
            pltpu.SemaphoreType.DMA((NSLOT,)),
        ],
        compiler_params=pltpu.CompilerParams(use_tc_tiling_on_sc=False),
    )
    def seg(edges_hbm, table_hbm, out_lo, out_hi,
            idxb, gath, zbuf, acc, isem, gsem, ssem):
        c = lax.axis_index("c")
        s = lax.axis_index("s")

        def idx_descs(slot, chunk0):
            return [pltpu.make_async_copy(
                edges_hbm.at[pl.ds(chunk0, G)], idxb.at[slot],
                isem.at[slot])]

        def transform_cols(slot):
            for j in range(G):
                for k in range(CH // 16):
                    sl = pl.ds(k * 16, 16)
                    v = idxb[slot, j, 1, sl]
                    idxb[slot, j, 1, sl] = v * 2 + c

        def fire_gathers(slot):
            for j in range(G):
                pltpu.async_copy(table_hbm.at[idxb.at[slot, j, 1]],
                                 gath.at[slot, j], gsem.at[slot])

        def drain_gathers(slot):
            for j in range(G):
                pltpu.make_async_copy(table_hbm.at[idxb.at[slot, j, 1]],
                                      gath.at[slot, j], gsem.at[slot]).wait()

        def fire_scatters(slot):
            for j in range(G):
                pltpu.async_copy(gath.at[slot, j], acc.at[idxb.at[slot, j, 0]],
                                 ssem.at[slot], add=True)

        def drain_scatters(slot):
            for j in range(G):
                pltpu.make_async_copy(gath.at[slot, j],
                                      acc.at[idxb.at[slot, j, 0]],
                                      ssem.at[slot]).wait()

        def run(out):
            chunk_base = s * CHUNKS_PER_TILE
            for q in range(NSLOT):
                for d in idx_descs(q, chunk_base + q * G):
                    d.start()
            for q in range(AG):
                for d in idx_descs(q, chunk_base + q * G):
                    d.wait()
                transform_cols(q)
                fire_gathers(q)

            def zb(i, carry):
                zbuf[i, :] = jnp.zeros((HALF,), jnp.float32)
                return carry

            lax.fori_loop(0, ZR, zb, 0)

            def za(k, carry):
                pltpu.sync_copy(
                    zbuf, acc.at[pl.ds(s * ROWS_PER_TILE + k * ZR, ZR)])
                return carry

            lax.fori_loop(0, ROWS_PER_TILE // ZR, za, 0)
            plsc.subcore_barrier()

            def grp(r, carry):
                p = lax.rem(r, NSLOT)
                pg = lax.rem(r + AG, NSLOT)
                pi = lax.rem(r + AI, NSLOT)
                drain_gathers(p)
                fire_scatters(p)

                @pl.when(r + AG < N_GROUPS)
                def _():
                    for d in idx_descs(pg, chunk_base + (r + AG) * G):
                        d.wait()
                    transform_cols(pg)
                    fire_gathers(pg)

                @pl.when(jnp.logical_and(r >= 2, r + AI < N_GROUPS))
                def _():
                    drain_scatters(pi)
                    for d in idx_descs(pi, chunk_base + (r + AI) * G):
                        d.start()

                return carry

            lax.fori_loop(0, N_GROUPS, grp, 0)
            for g in range(N_GROUPS - NSLOT, N_GROUPS):
                drain_scatters(g % NSLOT)

            def do_chunk(chunk_idx):
                pltpu.sync_copy(edges_hbm.at[pl.ds(chunk_idx, 1)],
                                idxb.at[0, pl.ds(0, 1)])
                for k in range(CH // 16):
                    sl = pl.ds(k * 16, 16)
                    v = idxb[0, 0, 1, sl]
                    idxb[0, 0, 1, sl] = v * 2 + c
                pltpu.async_copy(table_hbm.at[idxb.at[0, 0, 1]],
                                 gath.at[0, 0], gsem.at[0]).wait()
                pltpu.async_copy(gath.at[0, 0], acc.at[idxb.at[0, 0, 0]],
                                 ssem.at[0], add=True).wait()

            do_chunk(chunk_base + N_MAIN)

            @pl.when(s < 4)
            def _():
                do_chunk(16 * CHUNKS_PER_TILE + s)

            plsc.subcore_barrier()

            @pl.when(s < 15)
            def _():
                off = s * ROWS_PER_TILE
                pltpu.sync_copy(acc.at[pl.ds(off, ROWS_PER_TILE)],
                                out.at[pl.ds(off, ROWS_PER_TILE)])

            @pl.when(s == 15)
            def _():
                off = 15 * ROWS_PER_TILE
                rem = N_USERS - off
                pltpu.sync_copy(acc.at[pl.ds(off, rem)],
                                out.at[pl.ds(off, rem)])

        @pl.when(c == 0)
        def _():
            run(out_lo)

        @pl.when(c == 1)
        def _():
            run(out_hi)

    return seg(edges_il, table)


PBLK = 2048
OBLK = PBLK * 8


def _tc_finish_body(ep_lo_ref, ep_hi_ref, njp_ref, wb_lo_ref, wb_hi_ref,
                    s_ref, out_ref):
    acc = jnp.dot(ep_lo_ref[...], wb_lo_ref[...],
                  preferred_element_type=jnp.float32)
    acc += jnp.dot(ep_hi_ref[...], wb_hi_ref[...],
                   preferred_element_type=jnp.float32)
    scale = jnp.dot(njp_ref[...], s_ref[...],
                    preferred_element_type=jnp.float32)
    out_ref[...] = acc * scale


def _tc_finish(ep_lo, ep_hi, njp, mean_weight):
    grid = (pl.cdiv(PACK_ROWS, PBLK),)
    eye8 = jnp.eye(8, dtype=jnp.float32)
    wb_lo = jnp.kron(eye8, mean_weight[:HALF, :])
    wb_hi = jnp.kron(eye8, mean_weight[HALF:, :])
    sel = jnp.kron(eye8, jnp.ones((1, EMBED), jnp.float32))
    return pl.pallas_call(
        _tc_finish_body,
        grid=grid,
        in_specs=[
            pl.BlockSpec((PBLK, 128), lambda i: (i, 0)),
            pl.BlockSpec((PBLK, 128), lambda i: (i, 0)),
            pl.BlockSpec((PBLK, 8), lambda i: (i, 0)),
            pl.BlockSpec((128, 256), lambda i: (0, 0)),
            pl.BlockSpec((128, 256), lambda i: (0, 0)),
            pl.BlockSpec((8, 256), lambda i: (0, 0)),
        ],
        out_specs=pl.BlockSpec((PBLK, 256), lambda i: (i, 0)),
        out_shape=jax.ShapeDtypeStruct((PACK_ROWS, 256), jnp.float32),
    )(ep_lo, ep_hi, njp, wb_lo, wb_hi, sel)


def kernel(edge_index, user_n_j, item_n_j, user_emb, item_emb, mean_weight):
    edges_il = jnp.transpose(
        edge_index.astype(jnp.int32).reshape(2, N_CHUNKS, CH), (1, 0, 2))
    table = item_emb.reshape(2 * N_ITEMS, HALF)
    e_lo, e_hi = _sc_segment_sum(edges_il, table)
    ep_lo = e_lo.reshape(PACK_ROWS, 128)
    ep_hi = e_hi.reshape(PACK_ROWS, 128)
    njp = user_n_j.reshape(PACK_ROWS, 8)
    out_pack = _tc_finish(ep_lo, ep_hi, njp, mean_weight)
    return out_pack.reshape(N_USERS, EMBED)


# speedup vs baseline: 1.2942x; 1.2942x over previous
import functools

import jax
import jax.numpy as jnp
from jax import lax
from jax.experimental import pallas as pl
from jax.experimental.pallas import tpu as pltpu
from jax.experimental.pallas import tpu_sc as plsc

N_USERS = 100000
N_ITEMS = 100000
EMBED = 32
HALF = 16

CH = 128
G = 2
NSLOT = 6
AG = 3
AI = NSLOT - 1
ZR = 64
N_EDGE = 1600000
N_CHUNKS = N_EDGE // CH
CHUNKS_PER_TILE = N_CHUNKS // 16
N_MAIN = CHUNKS_PER_TILE - 1
N_GROUPS = N_MAIN // G
ROWS_PER_TILE = 6272
ACC_ROWS = ROWS_PER_TILE * 16
PACK_ROWS = N_USERS // 8


def _sc_segment_sum(edges_il, table):
    mesh = plsc.VectorSubcoreMesh(core_axis_name="c", subcore_axis_name="s")

    @functools.partial(
        pl.kernel,
        out_type=(
            jax.ShapeDtypeStruct((N_USERS, HALF), jnp.float32),
            jax.ShapeDtypeStruct((N_USERS, HALF), jnp.float32),
        ),
        mesh=mesh,
        scratch_types=[
            pltpu.VMEM((NSLOT, G, 2, CH), jnp.int32),
            pltpu.VMEM((NSLOT, G, CH, HALF), jnp.float32),
            pltpu.VMEM((ZR, HALF), jnp.float32),
            pltpu.VMEM_SHARED((ACC_ROWS, HALF), jnp.float32),
            pltpu.SemaphoreType.DMA((NSLOT,)),
            pltpu.SemaphoreType.DMA((NSLOT,)),
            pltpu.SemaphoreType.DMA((NSLOT,)),
        ],
        compiler_params=pltpu.CompilerParams(use_tc_tiling_on_sc=False),
    )
    def seg(edges_hbm, table_hbm, out_lo, out_hi,
            idxb, gath, zbuf, acc, isem, gsem, ssem):
        c = lax.axis_index("c")
        s = lax.axis_index("s")

        def idx_descs(slot, chunk0):
            return [pltpu.make_async_copy(
                edges_hbm.at[pl.ds(chunk0, G)], idxb.at[slot],
                isem.at[slot])]

        def transform_cols(slot):
            for j in range(G):
                for k in range(CH // 16):
                    sl = pl.ds(k * 16, 16)
                    v = idxb[slot, j, 1, sl]
                    idxb[slot, j, 1, sl] = v * 2 + c

        def fire_gathers(slot):
            for j in range(G):
                pltpu.async_copy(table_hbm.at[idxb.at[slot, j, 1]],
                                 gath.at[slot, j], gsem.at[slot])

        def drain_gathers(slot):
            for j in range(G):
                pltpu.make_async_copy(table_hbm.at[idxb.at[slot, j, 1]],
                                      gath.at[slot, j], gsem.at[slot]).wait()

        def fire_scatters(slot):
            for j in range(G):
                pltpu.async_copy(gath.at[slot, j], acc.at[idxb.at[slot, j, 0]],
                                 ssem.at[slot], add=True)

        def drain_scatters(slot):
            for j in range(G):
                pltpu.make_async_copy(gath.at[slot, j],
                                      acc.at[idxb.at[slot, j, 0]],
                                      ssem.at[slot]).wait()

        def run(out):
            chunk_base = s * CHUNKS_PER_TILE
            for q in range(NSLOT):
                for d in idx_descs(q, chunk_base + q * G):
                    d.start()
            for q in range(AG):
                for d in idx_descs(q, chunk_base + q * G):
                    d.wait()
                transform_cols(q)
                fire_gathers(q)

            def zb(i, carry):
                zbuf[i, :] = jnp.zeros((HALF,), jnp.float32)
                return carry

            lax.fori_loop(0, ZR, zb, 0)

            def za(k, carry):
                pltpu.sync_copy(
                    zbuf, acc.at[pl.ds(s * ROWS_PER_TILE + k * ZR, ZR)])
                return carry

            lax.fori_loop(0, ROWS_PER_TILE // ZR, za, 0)
            plsc.subcore_barrier()

            def grp(r, carry):
                p = lax.rem(r, NSLOT)
                pg = lax.rem(r + AG, NSLOT)
                pi = lax.rem(r + AI, NSLOT)
                drain_gathers(p)
                fire_scatters(p)

                @pl.when(r + AG < N_GROUPS)
                def _():
                    for d in idx_descs(pg, chunk_base + (r + AG) * G):
                        d.wait()
                    transform_cols(pg)
                    fire_gathers(pg)

                @pl.when(jnp.logical_and(r >= 1, r + AI < N_GROUPS))
                def _():
                    drain_scatters(pi)
                    for d in idx_descs(pi, chunk_base + (r + AI) * G):
                        d.start()

                return carry

            lax.fori_loop(0, N_GROUPS, grp, 0)
            for g in range(N_GROUPS - NSLOT, N_GROUPS):
                drain_scatters(g % NSLOT)

            def do_chunk(chunk_idx):
                pltpu.sync_copy(edges_hbm.at[pl.ds(chunk_idx, 1)],
                                idxb.at[0, pl.ds(0, 1)])
                for k in range(CH // 16):
                    sl = pl.ds(k * 16, 16)
                    v = idxb[0, 0, 1, sl]
                    idxb[0, 0, 1, sl] = v * 2 + c
                pltpu.async_copy(table_hbm.at[idxb.at[0, 0, 1]],
                                 gath.at[0, 0], gsem.at[0]).wait()
                pltpu.async_copy(gath.at[0, 0], acc.at[idxb.at[0, 0, 0]],
                                 ssem.at[0], add=True).wait()

            do_chunk(chunk_base + N_MAIN)

            @pl.when(s < 4)
            def _():
                do_chunk(16 * CHUNKS_PER_TILE + s)

            plsc.subcore_barrier()

            @pl.when(s < 15)
            def _():
                off = s * ROWS_PER_TILE
                pltpu.sync_copy(acc.at[pl.ds(off, ROWS_PER_TILE)],
                                out.at[pl.ds(off, ROWS_PER_TILE)])

            @pl.when(s == 15)
            def _():
                off = 15 * ROWS_PER_TILE
                rem = N_USERS - off
                pltpu.sync_copy(acc.at[pl.ds(off, rem)],
                                out.at[pl.ds(off, rem)])

        @pl.when(c == 0)
        def _():
            run(out_lo)

        @pl.when(c == 1)
        def _():
            run(out_hi)

    return seg(edges_il, table)


PBLK = 2048
OBLK = PBLK * 8


def _tc_finish_body(ep_lo_ref, ep_hi_ref, njp_ref, wb_lo_ref, wb_hi_ref,
                    s_ref, out_ref):
    acc = jnp.dot(ep_lo_ref[...], wb_lo_ref[...],
                  preferred_element_type=jnp.float32)
    acc += jnp.dot(ep_hi_ref[...], wb_hi_ref[...],
                   preferred_element_type=jnp.float32)
    scale = jnp.dot(njp_ref[...], s_ref[...],
                    preferred_element_type=jnp.float32)
    out_ref[...] = acc * scale


def _tc_finish(ep_lo, ep_hi, njp, mean_weight):
    grid = (pl.cdiv(PACK_ROWS, PBLK),)
    eye8 = jnp.eye(8, dtype=jnp.float32)
    wb_lo = jnp.kron(eye8, mean_weight[:HALF, :])
    wb_hi = jnp.kron(eye8, mean_weight[HALF:, :])
    sel = jnp.kron(eye8, jnp.ones((1, EMBED), jnp.float32))
    return pl.pallas_call(
        _tc_finish_body,
        grid=grid,
        in_specs=[
            pl.BlockSpec((PBLK, 128), lambda i: (i, 0)),
            pl.BlockSpec((PBLK, 128), lambda i: (i, 0)),
            pl.BlockSpec((PBLK, 8), lambda i: (i, 0)),
            pl.BlockSpec((128, 256), lambda i: (0, 0)),
            pl.BlockSpec((128, 256), lambda i: (0, 0)),
            pl.BlockSpec((8, 256), lambda i: (0, 0)),
        ],
        out_specs=pl.BlockSpec((PBLK, 256), lambda i: (i, 0)),
        out_shape=jax.ShapeDtypeStruct((PACK_ROWS, 256), jnp.float32),
    )(ep_lo, ep_hi, njp, wb_lo, wb_hi, sel)


def kernel(edge_index, user_n_j, item_n_j, user_emb, item_emb, mean_weight):
    edges_il = jnp.transpose(
        edge_index.astype(jnp.int32).reshape(2, N_CHUNKS, CH), (1, 0, 2))
    table = item_emb.reshape(2 * N_ITEMS, HALF)
    e_lo, e_hi = _sc_segment_sum(edges_il, table)
    ep_lo = e_lo.reshape(PACK_ROWS, 128)
    ep_hi = e_hi.reshape(PACK_ROWS, 128)
    njp = user_n_j.reshape(PACK_ROWS, 8)
    out_pack = _tc_finish(ep_lo, ep_hi, njp, mean_weight)
    return out_pack.reshape(N_USERS, EMBED)


# speedup vs baseline: 1.3063x; 1.0093x over previous
import functools

import jax
import jax.numpy as jnp
from jax import lax
from jax.experimental import pallas as pl
from jax.experimental.pallas import tpu as pltpu
from jax.experimental.pallas import tpu_sc as plsc

N_USERS = 100000
N_ITEMS = 100000
EMBED = 32
HALF = 16

CH = 128
G = 2
NSLOT = 6
AG = 3
AI = NSLOT - 1
ZR = 64
N_EDGE = 1600000
N_CHUNKS = N_EDGE // CH
CHUNKS_PER_TILE = N_CHUNKS // 16
N_MAIN = CHUNKS_PER_TILE - 1
N_GROUPS = N_MAIN // G
ROWS_PER_TILE = 6272
ACC_ROWS = ROWS_PER_TILE * 16
PACK_ROWS = N_USERS // 8


def _sc_segment_sum(edges_il, table):
    mesh = plsc.VectorSubcoreMesh(core_axis_name="c", subcore_axis_name="s")

    @functools.partial(
        pl.kernel,
        out_type=(
            jax.ShapeDtypeStruct((N_USERS, HALF), jnp.float32),
            jax.ShapeDtypeStruct((N_USERS, HALF), jnp.float32),
        ),
        mesh=mesh,
        scratch_types=[
            pltpu.VMEM((NSLOT, G, 2, CH), jnp.int32),
            pltpu.VMEM((NSLOT, G, CH, HALF), jnp.float32),
            pltpu.VMEM((ZR, HALF), jnp.float32),
            pltpu.VMEM_SHARED((ACC_ROWS, HALF), jnp.float32),
            pltpu.SemaphoreType.DMA((NSLOT,)),
            pltpu.SemaphoreType.DMA((NSLOT,)),
            pltpu.SemaphoreType.DMA((NSLOT,)),
        ],
        compiler_params=pltpu.CompilerParams(use_tc_tiling_on_sc=False),
    )
    def seg(edges_hbm, table_hbm, out_lo, out_hi,
            idxb, gath, zbuf, acc, isem, gsem, ssem):
        c = lax.axis_index("c")
        s = lax.axis_index("s")

        def idx_descs(slot, chunk0):
            return [pltpu.make_async_copy(
                edges_hbm.at[pl.ds(chunk0, G)], idxb.at[slot],
                isem.at[slot])]

        def transform_cols(slot):
            for j in range(G):
                for k in range(CH // 16):
                    sl = pl.ds(k * 16, 16)
                    v = idxb[slot, j, 1, sl]
                    idxb[slot, j, 1, sl] = v * 2 + c

        def fire_gathers(slot):
            for j in range(G):
                pltpu.async_copy(table_hbm.at[idxb.at[slot, j, 1]],
                                 gath.at[slot, j], gsem.at[slot])

        def drain_gathers(slot):
            for j in range(G):
                pltpu.make_async_copy(table_hbm.at[idxb.at[slot, j, 1]],
                                      gath.at[slot, j], gsem.at[slot]).wait()

        def fire_scatters(slot):
            for j in range(G):
                pltpu.async_copy(gath.at[slot, j], acc.at[idxb.at[slot, j, 0]],
                                 ssem.at[slot], add=True)

        def drain_scatters(slot):
            for j in range(G):
                pltpu.make_async_copy(gath.at[slot, j],
                                      acc.at[idxb.at[slot, j, 0]],
                                      ssem.at[slot]).wait()

        def run(out):
            chunk_base = s * CHUNKS_PER_TILE
            for q in range(NSLOT):
                for d in idx_descs(q, chunk_base + q * G):
                    d.start()
            for q in range(AG):
                for d in idx_descs(q, chunk_base + q * G):
                    d.wait()
                transform_cols(q)
                fire_gathers(q)

            def zb(i, carry):
                zbuf[i, :] = jnp.zeros((HALF,), jnp.float32)
                return carry

            lax.fori_loop(0, ZR, zb, 0)

            def za(k, carry):
                pltpu.sync_copy(
                    zbuf, acc.at[pl.ds(s * ROWS_PER_TILE + k * ZR, ZR)])
                return carry

            lax.fori_loop(0, ROWS_PER_TILE // ZR, za, 0)
            plsc.subcore_barrier()

            def grp(r, carry):
                p = lax.rem(r, NSLOT)
                pg = lax.rem(r + AG, NSLOT)
                pi = lax.rem(r + AI, NSLOT)
                @pl.when(r + AG < N_GROUPS)
                def _():
                    for d in idx_descs(pg, chunk_base + (r + AG) * G):
                        d.wait()
                    transform_cols(pg)
                    fire_gathers(pg)

                drain_gathers(p)
                fire_scatters(p)

                @pl.when(jnp.logical_and(r >= 1, r + AI < N_GROUPS))
                def _():
                    drain_scatters(pi)
                    for d in idx_descs(pi, chunk_base + (r + AI) * G):
                        d.start()

                return carry

            lax.fori_loop(0, N_GROUPS, grp, 0)
            for g in range(N_GROUPS - NSLOT, N_GROUPS):
                drain_scatters(g % NSLOT)

            def do_chunk(chunk_idx):
                pltpu.sync_copy(edges_hbm.at[pl.ds(chunk_idx, 1)],
                                idxb.at[0, pl.ds(0, 1)])
                for k in range(CH // 16):
                    sl = pl.ds(k * 16, 16)
                    v = idxb[0, 0, 1, sl]
                    idxb[0, 0, 1, sl] = v * 2 + c
                pltpu.async_copy(table_hbm.at[idxb.at[0, 0, 1]],
                                 gath.at[0, 0], gsem.at[0]).wait()
                pltpu.async_copy(gath.at[0, 0], acc.at[idxb.at[0, 0, 0]],
                                 ssem.at[0], add=True).wait()

            do_chunk(chunk_base + N_MAIN)

            @pl.when(s < 4)
            def _():
                do_chunk(16 * CHUNKS_PER_TILE + s)

            plsc.subcore_barrier()

            @pl.when(s < 15)
            def _():
                off = s * ROWS_PER_TILE
                pltpu.sync_copy(acc.at[pl.ds(off, ROWS_PER_TILE)],
                                out.at[pl.ds(off, ROWS_PER_TILE)])

            @pl.when(s == 15)
            def _():
                off = 15 * ROWS_PER_TILE
                rem = N_USERS - off
                pltpu.sync_copy(acc.at[pl.ds(off, rem)],
                                out.at[pl.ds(off, rem)])

        @pl.when(c == 0)
        def _():
            run(out_lo)

        @pl.when(c == 1)
        def _():
            run(out_hi)

    return seg(edges_il, table)


PBLK = 2048
OBLK = PBLK * 8


def _tc_finish_body(ep_lo_ref, ep_hi_ref, njp_ref, wb_lo_ref, wb_hi_ref,
                    s_ref, out_ref):
    acc = jnp.dot(ep_lo_ref[...], wb_lo_ref[...],
                  preferred_element_type=jnp.float32)
    acc += jnp.dot(ep_hi_ref[...], wb_hi_ref[...],
                   preferred_element_type=jnp.float32)
    scale = jnp.dot(njp_ref[...], s_ref[...],
                    preferred_element_type=jnp.float32)
    out_ref[...] = acc * scale


def _tc_finish(ep_lo, ep_hi, njp, mean_weight):
    grid = (pl.cdiv(PACK_ROWS, PBLK),)
    eye8 = jnp.eye(8, dtype=jnp.float32)
    wb_lo = jnp.kron(eye8, mean_weight[:HALF, :])
    wb_hi = jnp.kron(eye8, mean_weight[HALF:, :])
    sel = jnp.kron(eye8, jnp.ones((1, EMBED), jnp.float32))
    return pl.pallas_call(
        _tc_finish_body,
        grid=grid,
        in_specs=[
            pl.BlockSpec((PBLK, 128), lambda i: (i, 0)),
            pl.BlockSpec((PBLK, 128), lambda i: (i, 0)),
            pl.BlockSpec((PBLK, 8), lambda i: (i, 0)),
            pl.BlockSpec((128, 256), lambda i: (0, 0)),
            pl.BlockSpec((128, 256), lambda i: (0, 0)),
            pl.BlockSpec((8, 256), lambda i: (0, 0)),
        ],
        out_specs=pl.BlockSpec((PBLK, 256), lambda i: (i, 0)),
        out_shape=jax.ShapeDtypeStruct((PACK_ROWS, 256), jnp.float32),
    )(ep_lo, ep_hi, njp, wb_lo, wb_hi, sel)


def kernel(edge_index, user_n_j, item_n_j, user_emb, item_emb, mean_weight):
    edges_il = jnp.transpose(
        edge_index.astype(jnp.int32).reshape(2, N_CHUNKS, CH), (1, 0, 2))
    table = item_emb.reshape(2 * N_ITEMS, HALF)
    e_lo, e_hi = _sc_segment_sum(edges_il, table)
    ep_lo = e_lo.reshape(PACK_ROWS, 128)
    ep_hi = e_hi.reshape(PACK_ROWS, 128)
    njp = user_n_j.reshape(PACK_ROWS, 8)
    out_pack = _tc_finish(ep_lo, ep_hi, njp, mean_weight)
    return out_pack.reshape(N_USERS, EMBED)


# speedup vs baseline: 1.3266x; 1.0155x over previous
import functools

import jax
import jax.numpy as jnp
from jax import lax
from jax.experimental import pallas as pl
from jax.experimental.pallas import tpu as pltpu
from jax.experimental.pallas import tpu_sc as plsc

N_USERS = 100000
N_ITEMS = 100000
EMBED = 32
HALF = 16

CH = 128
G = 2
NSLOT = 6
AG = 3
AI = NSLOT - 1
ZR = 128
N_EDGE = 1600000
N_CHUNKS = N_EDGE // CH
CHUNKS_PER_TILE = N_CHUNKS // 16
N_MAIN = CHUNKS_PER_TILE - 1
N_GROUPS = N_MAIN // G
ROWS_PER_TILE = 6272
ACC_ROWS = ROWS_PER_TILE * 16
PACK_ROWS = N_USERS // 8


def _sc_segment_sum(edges_il, table):
    mesh = plsc.VectorSubcoreMesh(core_axis_name="c", subcore_axis_name="s")

    @functools.partial(
        pl.kernel,
        out_type=(
            jax.ShapeDtypeStruct((N_USERS, HALF), jnp.float32),
            jax.ShapeDtypeStruct((N_USERS, HALF), jnp.float32),
        ),
        mesh=mesh,
        scratch_types=[
            pltpu.VMEM((NSLOT, G, 2, CH), jnp.int32),
            pltpu.VMEM((NSLOT, G, CH, HALF), jnp.float32),
            pltpu.VMEM((ZR, HALF), jnp.float32),
            pltpu.VMEM_SHARED((ACC_ROWS, HALF), jnp.float32),
            pltpu.SemaphoreType.DMA((NSLOT,)),
            pltpu.SemaphoreType.DMA((NSLOT,)),
            pltpu.SemaphoreType.DMA((NSLOT,)),
            pltpu.SemaphoreType.DMA,
        ],
        compiler_params=pltpu.CompilerParams(use_tc_tiling_on_sc=False),
    )
    def seg(edges_hbm, table_hbm, out_lo, out_hi,
            idxb, gath, zbuf, acc, isem, gsem, ssem, zsem):
        c = lax.axis_index("c")
        s = lax.axis_index("s")

        def idx_descs(slot, chunk0):
            return [pltpu.make_async_copy(
                edges_hbm.at[pl.ds(chunk0, G)], idxb.at[slot],
                isem.at[slot])]

        def transform_cols(slot):
            for j in range(G):
                for k in range(CH // 16):
                    sl = pl.ds(k * 16, 16)
                    v = idxb[slot, j, 1, sl]
                    idxb[slot, j, 1, sl] = v * 2 + c

        def fire_gathers(slot):
            for j in range(G):
                pltpu.async_copy(table_hbm.at[idxb.at[slot, j, 1]],
                                 gath.at[slot, j], gsem.at[slot])

        def drain_gathers(slot):
            for j in range(G):
                pltpu.make_async_copy(table_hbm.at[idxb.at[slot, j, 1]],
                                      gath.at[slot, j], gsem.at[slot]).wait()

        def fire_scatters(slot):
            for j in range(G):
                pltpu.async_copy(gath.at[slot, j], acc.at[idxb.at[slot, j, 0]],
                                 ssem.at[slot], add=True)

        def drain_scatters(slot):
            for j in range(G):
                pltpu.make_async_copy(gath.at[slot, j],
                                      acc.at[idxb.at[slot, j, 0]],
                                      ssem.at[slot]).wait()

        def run(out):
            chunk_base = s * CHUNKS_PER_TILE
            for q in range(NSLOT):
                for d in idx_descs(q, chunk_base + q * G):
                    d.start()
            for q in range(AG):
                for d in idx_descs(q, chunk_base + q * G):
                    d.wait()
                transform_cols(q)
                fire_gathers(q)

            def zb(i, carry):
                zbuf[i, :] = jnp.zeros((HALF,), jnp.float32)
                return carry

            lax.fori_loop(0, ZR, zb, 0)

            def za(k, carry):
                pltpu.async_copy(
                    zbuf, acc.at[pl.ds(s * ROWS_PER_TILE + k * ZR, ZR)], zsem)
                return carry

            lax.fori_loop(0, ROWS_PER_TILE // ZR, za, 0)

            def zw(k, carry):
                pltpu.make_async_copy(
                    zbuf, acc.at[pl.ds(s * ROWS_PER_TILE + k * ZR, ZR)],
                    zsem).wait()
                return carry

            lax.fori_loop(0, ROWS_PER_TILE // ZR, zw, 0)
            plsc.subcore_barrier()

            def grp(r, carry):
                p = lax.rem(r, NSLOT)
                pg = lax.rem(r + AG, NSLOT)
                pi = lax.rem(r + AI, NSLOT)
                @pl.when(r + AG < N_GROUPS)
                def _():
                    for d in idx_descs(pg, chunk_base + (r + AG) * G):
                        d.wait()
                    transform_cols(pg)
                    fire_gathers(pg)

                drain_gathers(p)
                fire_scatters(p)

                @pl.when(jnp.logical_and(r >= 1, r + AI < N_GROUPS))
                def _():
                    drain_scatters(pi)
                    for d in idx_descs(pi, chunk_base + (r + AI) * G):
                        d.start()

                return carry

            lax.fori_loop(0, N_GROUPS, grp, 0)
            for g in range(N_GROUPS - NSLOT, N_GROUPS):
                drain_scatters(g % NSLOT)

            def do_chunk(chunk_idx):
                pltpu.sync_copy(edges_hbm.at[pl.ds(chunk_idx, 1)],
                                idxb.at[0, pl.ds(0, 1)])
                for k in range(CH // 16):
                    sl = pl.ds(k * 16, 16)
                    v = idxb[0, 0, 1, sl]
                    idxb[0, 0, 1, sl] = v * 2 + c
                pltpu.async_copy(table_hbm.at[idxb.at[0, 0, 1]],
                                 gath.at[0, 0], gsem.at[0]).wait()
                pltpu.async_copy(gath.at[0, 0], acc.at[idxb.at[0, 0, 0]],
                                 ssem.at[0], add=True).wait()

            do_chunk(chunk_base + N_MAIN)

            @pl.when(s < 4)
            def _():
                do_chunk(16 * CHUNKS_PER_TILE + s)

            plsc.subcore_barrier()

            @pl.when(s < 15)
            def _():
                off = s * ROWS_PER_TILE
                pltpu.sync_copy(acc.at[pl.ds(off, ROWS_PER_TILE)],
                                out.at[pl.ds(off, ROWS_PER_TILE)])

            @pl.when(s == 15)
            def _():
                off = 15 * ROWS_PER_TILE
                rem = N_USERS - off
                pltpu.sync_copy(acc.at[pl.ds(off, rem)],
                                out.at[pl.ds(off, rem)])

        @pl.when(c == 0)
        def _():
            run(out_lo)

        @pl.when(c == 1)
        def _():
            run(out_hi)

    return seg(edges_il, table)


PBLK = 2048
OBLK = PBLK * 8


def _tc_finish_body(ep_lo_ref, ep_hi_ref, njp_ref, wb_lo_ref, wb_hi_ref,
                    s_ref, out_ref):
    acc = jnp.dot(ep_lo_ref[...], wb_lo_ref[...],
                  preferred_element_type=jnp.float32)
    acc += jnp.dot(ep_hi_ref[...], wb_hi_ref[...],
                   preferred_element_type=jnp.float32)
    scale = jnp.dot(njp_ref[...], s_ref[...],
                    preferred_element_type=jnp.float32)
    out_ref[...] = acc * scale


def _tc_finish(ep_lo, ep_hi, njp, mean_weight):
    grid = (pl.cdiv(PACK_ROWS, PBLK),)
    eye8 = jnp.eye(8, dtype=jnp.float32)
    wb_lo = jnp.kron(eye8, mean_weight[:HALF, :])
    wb_hi = jnp.kron(eye8, mean_weight[HALF:, :])
    sel = jnp.kron(eye8, jnp.ones((1, EMBED), jnp.float32))
    return pl.pallas_call(
        _tc_finish_body,
        grid=grid,
        in_specs=[
            pl.BlockSpec((PBLK, 128), lambda i: (i, 0)),
            pl.BlockSpec((PBLK, 128), lambda i: (i, 0)),
            pl.BlockSpec((PBLK, 8), lambda i: (i, 0)),
            pl.BlockSpec((128, 256), lambda i: (0, 0)),
            pl.BlockSpec((128, 256), lambda i: (0, 0)),
            pl.BlockSpec((8, 256), lambda i: (0, 0)),
        ],
        out_specs=pl.BlockSpec((PBLK, 256), lambda i: (i, 0)),
        out_shape=jax.ShapeDtypeStruct((PACK_ROWS, 256), jnp.float32),
    )(ep_lo, ep_hi, njp, wb_lo, wb_hi, sel)


def kernel(edge_index, user_n_j, item_n_j, user_emb, item_emb, mean_weight):
    edges_il = jnp.transpose(
        edge_index.astype(jnp.int32).reshape(2, N_CHUNKS, CH), (1, 0, 2))
    table = item_emb.reshape(2 * N_ITEMS, HALF)
    e_lo, e_hi = _sc_segment_sum(edges_il, table)
    ep_lo = e_lo.reshape(PACK_ROWS, 128)
    ep_hi = e_hi.reshape(PACK_ROWS, 128)
    njp = user_n_j.reshape(PACK_ROWS, 8)
    out_pack = _tc_finish(ep_lo, ep_hi, njp, mean_weight)
    return out_pack.reshape(N_USERS, EMBED)
